# baseline (speedup 1.0000x reference)
import functools

import jax
import jax.numpy as jnp
from jax import lax
from jax.experimental import pallas as pl
from jax.experimental.pallas import tpu as pltpu
from jax.experimental.pallas import tpu_sc as plsc

N = 10000
E = 320000
IN_FEAT = 128
REL_DIM = 32
HID = 64
NUM_RELS = 200
RELS_PAD = 256
NUM_BASES = 4
DP = 256

NC = 2
NS = 16
NW = NC * NS
CH = 128
EPT = 10240
NCH = EPT // CH
EPAD = NW * EPT
_EBLK = 2048
_EGRID = EPAD // _EBLK
_EGRID_H = _EGRID // 2
NAG = 10240
NPK2 = NAG // 2
RPT2 = NPK2 // NS

_MESH = plsc.VectorSubcoreMesh(core_axis_name="c", subcore_axis_name="s")


def _gather_body(h_hbm, src_hbm, out_hbm, idx_v, rows_v, sem):
    c = lax.axis_index("c")
    s = lax.axis_index("s")
    wid = s * NC + c
    base = wid * EPT

    @pl.loop(0, NCH)
    def _chunk(i):
        off = pl.multiple_of(base + i * CH, CH)
        pltpu.sync_copy(src_hbm.at[pl.ds(off, CH)], idx_v)
        pltpu.async_copy(h_hbm.at[idx_v], rows_v, sem).wait()
        pltpu.sync_copy(rows_v, out_hbm.at[pl.ds(off, CH)])


_sc_gather = functools.partial(
    pl.kernel,
    mesh=_MESH,
    out_type=jax.ShapeDtypeStruct((EPAD, DP), jnp.float32),
    scratch_types=[
        pltpu.VMEM((CH,), jnp.int32),
        pltpu.VMEM((CH, DP), jnp.float32),
        pltpu.SemaphoreType.DMA,
    ],
)(_gather_body)


def _tc_scatter_body(dst_smem, msg_ref, out_ref, acc_ref):
    i = pl.program_id(0)

    @pl.when(i == 0)
    def _init():
        acc_ref[...] = jnp.zeros_like(acc_ref)

    base = i * _EBLK

    def step(j, carry):
        d = dst_smem[base + j]
        acc_ref[pl.ds(d, 1), :] = (acc_ref[pl.ds(d, 1), :]
                                   + msg_ref[pl.ds(j, 1), :])
        return carry

    lax.fori_loop(0, _EBLK, step, 0)

    @pl.when(i == _EGRID_H - 1)
    def _out():
        out_ref[...] = acc_ref[...]


def _tc_scatter(dst_half, msg_half):
    return pl.pallas_call(
        _tc_scatter_body,
        grid_spec=pltpu.PrefetchScalarGridSpec(
            num_scalar_prefetch=1,
            grid=(_EGRID_H,),
            in_specs=[pl.BlockSpec((_EBLK, HID), lambda i, d: (i, 0))],
            out_specs=pl.BlockSpec((NAG, HID), lambda i, d: (0, 0)),
            scratch_shapes=[pltpu.VMEM((NAG, HID), jnp.float32)],
        ),
        out_shape=jax.ShapeDtypeStruct((NAG, HID), jnp.float32),
    )(dst_half, msg_half)


_NBLK = 10
_NR = N // _NBLK


def _input_body(feat_ref, qr_ref, rel_ref, wf_ref, we_ref, bin_ref, awe_ref,
                h0_ref, era_ref):
    t_in = jnp.dot(rel_ref[...], we_ref[...], preferred_element_type=jnp.float32)
    t_attn = jnp.dot(rel_ref[...], awe_ref[...], preferred_element_type=jnp.float32)
    iota = lax.broadcasted_iota(jnp.int32, (_NR, RELS_PAD), 1)
    onehot = (qr_ref[...] == iota).astype(jnp.float32)
    er_in = jnp.dot(onehot, t_in, preferred_element_type=jnp.float32)
    era_ref[...] = jnp.dot(onehot, t_attn, preferred_element_type=jnp.float32)
    h0 = jnp.dot(feat_ref[...], wf_ref[...], preferred_element_type=jnp.float32)
    h0_ref[...] = jnp.maximum(h0 + er_in + bin_ref[...], 0.0)


def _tc_input(feat, qr2d, rel_pad, w_f, w_e, b_in, attn_we):
    return pl.pallas_call(
        _input_body,
        grid=(_NBLK,),
        in_specs=[
            pl.BlockSpec((_NR, IN_FEAT), lambda i: (i, 0)),
            pl.BlockSpec((_NR, 1), lambda i: (i, 0)),
            pl.BlockSpec((RELS_PAD, REL_DIM), lambda i: (0, 0)),
            pl.BlockSpec((IN_FEAT, HID), lambda i: (0, 0)),
            pl.BlockSpec((REL_DIM, HID), lambda i: (0, 0)),
            pl.BlockSpec((1, HID), lambda i: (0, 0)),
            pl.BlockSpec((REL_DIM, HID), lambda i: (0, 0)),
        ],
        out_specs=[
            pl.BlockSpec((_NR, HID), lambda i: (i, 0)),
            pl.BlockSpec((_NR, HID), lambda i: (i, 0)),
        ],
        out_shape=[
            jax.ShapeDtypeStruct((N, HID), jnp.float32),
            jax.ShapeDtypeStruct((N, HID), jnp.float32),
        ],
    )(feat, qr2d, rel_pad, w_f, w_e, b_in, attn_we)


def _msg_body(hs_ref, et_ref, bmat_ref, coef_ref, msg_ref):
    hsrc = hs_ref[:, :HID]
    hb = jnp.dot(hsrc, bmat_ref[...], preferred_element_type=jnp.float32)
    iota = lax.broadcasted_iota(jnp.int32, (_EBLK, RELS_PAD), 1)
    onehot = (et_ref[...] == iota).astype(jnp.float32)
    cb = jnp.dot(onehot, coef_ref[...], preferred_element_type=jnp.float32)
    acc = cb[:, 0:1] * hb[:, 0:HID]
    for b in range(1, NUM_BASES):
        acc = acc + cb[:, b:b + 1] * hb[:, b * HID:(b + 1) * HID]
    msg_ref[...] = acc


def _tc_msg(h_srcp, et2d, bmat, coef_pad):
    return pl.pallas_call(
        _msg_body,
        grid=(_EGRID,),
        in_specs=[
            pl.BlockSpec((_EBLK, DP), lambda i: (i, 0)),
            pl.BlockSpec((_EBLK, 1), lambda i: (i, 0)),
            pl.BlockSpec((HID, NUM_BASES * HID), lambda i: (0, 0)),
            pl.BlockSpec((RELS_PAD, NUM_BASES), lambda i: (0, 0)),
        ],
        out_specs=pl.BlockSpec((_EBLK, HID), lambda i: (i, 0)),
        out_shape=jax.ShapeDtypeStruct((EPAD, HID), jnp.float32),
    )(h_srcp, et2d, bmat, coef_pad)


def _combine_body(p0_ref, p1_ref, h_ref, ws_ref, b_ref, out_ref):
    hs = jnp.dot(h_ref[...], ws_ref[...], preferred_element_type=jnp.float32)
    out_ref[...] = jnp.maximum(p0_ref[...] + p1_ref[...] + hs + b_ref[...], 0.0)


def _tc_combine(agg0, agg1, h, w_self, b_self):
    return pl.pallas_call(
        _combine_body,
        grid=(_NBLK,),
        in_specs=[
            pl.BlockSpec((_NR, HID), lambda i: (i, 0)),
            pl.BlockSpec((_NR, HID), lambda i: (i, 0)),
            pl.BlockSpec((_NR, HID), lambda i: (i, 0)),
            pl.BlockSpec((HID, HID), lambda i: (0, 0)),
            pl.BlockSpec((1, HID), lambda i: (0, 0)),
        ],
        out_specs=pl.BlockSpec((_NR, HID), lambda i: (i, 0)),
        out_shape=jax.ShapeDtypeStruct((N, HID), jnp.float32),
    )(agg0, agg1, h, w_self, b_self)


def _readout_body(h_ref, era_ref, awh_ref, ab_ref, sw_ref, sb_ref,
                  ow_ref, ob_ref, out_ref):
    h = h_ref[...]
    a = jnp.tanh(
        jnp.dot(h, awh_ref[...], preferred_element_type=jnp.float32)
        + era_ref[...] + ab_ref[...])
    s = jnp.sum(a * sw_ref[...], axis=1, keepdims=True) + sb_ref[0, 0]
    m = jnp.max(s)
    e = jnp.exp(s - m)
    alpha = e / jnp.sum(e)
    z = jnp.sum(h * alpha, axis=0, keepdims=True)
    out_ref[...] = (jnp.sum(z * ow_ref[...]) + ob_ref[0, 0]).reshape(1, 1)


def _tc_readout(h, era, attn_wh, attn_b, score_w, score_b, out_w, out_b):
    return pl.pallas_call(
        _readout_body,
        out_shape=jax.ShapeDtypeStruct((1, 1), jnp.float32),
    )(h, era, attn_wh, attn_b, score_w, score_b, out_w, out_b)


def kernel(feat, query_rel, edge_index, etypes, params):
    p = params
    pad_e = EPAD - E
    spread = jnp.arange(pad_e, dtype=jnp.int32) % N
    src_pad = jnp.concatenate([edge_index[0], spread])
    dst_pad = jnp.concatenate([edge_index[1], spread])
    et_pad = jnp.concatenate(
        [etypes, jnp.full((pad_e,), NUM_RELS, jnp.int32)]).reshape(EPAD, 1)
    qr2d = query_rel.reshape(N, 1)

    rel_pad = jnp.zeros((RELS_PAD, REL_DIM), jnp.float32).at[:NUM_RELS].set(
        p['rel_emb'])
    w_f = p['in_W'][:IN_FEAT]
    w_e = p['in_W'][IN_FEAT:]
    b_in = p['in_b'].reshape(1, HID)
    attn_wh = p['attn_W'][:HID]
    attn_we = p['attn_W'][HID:]
    attn_b = p['attn_b'].reshape(1, HID)
    score_w = p['score_W'].reshape(1, HID)
    score_b = p['score_b'].reshape(1, 1)
    out_w = p['out_W'].reshape(1, HID)
    out_b = p['out_b'].reshape(1, 1)

    h, era = _tc_input(feat, qr2d, rel_pad, w_f, w_e, b_in, attn_we)

    for l in range(2):
        bmat = p['basis%d' % l].transpose(1, 0, 2).reshape(HID, NUM_BASES * HID)
        coef_pad = jnp.zeros((RELS_PAD, NUM_BASES), jnp.float32).at[
            :NUM_RELS].set(p['coef%d' % l])
        hpad = jnp.zeros((N, DP), jnp.float32).at[:, :HID].set(h)
        h_srcp = _sc_gather(hpad, src_pad)
        msg = _tc_msg(h_srcp, et_pad, bmat, coef_pad)
        eh = EPAD // 2
        agg0 = _tc_scatter(dst_pad[:eh], msg[:eh])
        agg1 = _tc_scatter(dst_pad[eh:], msg[eh:])
        h = _tc_combine(agg0, agg1, h, p['self%d' % l],
                        p['sbias%d' % l].reshape(1, HID))

    out = _tc_readout(h, era, attn_wh, attn_b, score_w, score_b, out_w, out_b)
    return out.reshape(1)
